# baseline (speedup 1.0000x reference)
import functools

import jax
import jax.numpy as jnp
from jax import lax
from jax.experimental import pallas as pl
from jax.experimental.pallas import tpu as pltpu
from jax.experimental.pallas import tpu_sc as plsc

_ROWS = 16
_CW = 1024
_NSEG = 4
_SEG = _CW // _NSEG
_CHUNK = 20000

_KS0 = 0
_KS1 = 42
_KS2 = _KS0 ^ _KS1 ^ 0x1BD11BDA
_ROT = ((13, 15, 26, 6), (17, 29, 16, 24))


def _threefry_bits(p):
    ks = (jnp.int32(_KS0), jnp.int32(_KS1), jnp.int32(_KS2))
    x0 = jnp.zeros_like(p) + ks[0]
    x1 = p + ks[1]
    for i in range(5):
        for r in _ROT[i % 2]:
            x0 = x0 + x1
            x1 = (lax.shift_left(x1, jnp.int32(r))
                  | lax.shift_right_logical(x1, jnp.int32(32 - r)))
            x1 = x1 ^ x0
        x0 = x0 + ks[(i + 1) % 3]
        x1 = x1 + ks[(i + 2) % 3] + jnp.int32(i + 1)
    return x0 ^ x1


def _ordered_int(x):
    xi = lax.bitcast_convert_type(x, jnp.int32)
    return jnp.where(xi >= 0, xi, xi ^ jnp.int32(0x7FFFFFFF))


def _kth_largest(o, keff, nbits=32):
    r = o.shape[0]
    min32 = jnp.int32(-2147483648)

    def bit_step(j, prefix):
        cand = prefix | lax.shift_left(jnp.int32(1), jnp.int32(31) - j)
        thr = cand ^ min32
        cnt = jnp.sum((o >= thr).astype(jnp.int32), axis=1, keepdims=True)
        return jnp.where(cnt >= keff, cand, prefix)

    prefix = lax.fori_loop(0, nbits, bit_step, jnp.zeros((r, 1), jnp.int32))
    return prefix ^ min32


def _prefilter_body(keff_ref, x_ref, tgm_ref):
    keff = keff_ref[0, 0]
    x = x_ref[...]
    v = x.shape[1]
    q = v // 8
    parts = [x[:, i * q:(i + 1) * q] for i in range(8)]
    while len(parts) > 1:
        parts = [jnp.maximum(parts[i], parts[i + 1])
                 for i in range(0, len(parts), 2)]
    t_gm = _kth_largest(_ordered_int(parts[0]), keff, nbits=20)
    t_f = lax.bitcast_convert_type(
        jnp.where(t_gm >= 0, t_gm, t_gm ^ jnp.int32(0x7FFFFFFF)),
        jnp.float32)
    tgm_ref[...] = jnp.broadcast_to(t_f, tgm_ref.shape)


def _make_compactor(b, v, n_workers):
    rows_per_w = b // n_workers
    n_chunks = v // _CHUNK
    vregs = _CHUNK // 16
    main = vregs // _NSEG
    rem = vregs - main * _NSEG
    mesh = plsc.VectorSubcoreMesh(core_axis_name="c", subcore_axis_name="s")

    @functools.partial(
        pl.kernel, mesh=mesh,
        compiler_params=pltpu.CompilerParams(needs_layout_passes=False),
        out_type=[
            jax.ShapeDtypeStruct((b * _CW,), jnp.int32),
            jax.ShapeDtypeStruct((b * _CW,), jnp.float32),
        ],
        scratch_types=[
            pltpu.VMEM((_CHUNK,), jnp.float32),
            pltpu.VMEM((_CHUNK,), jnp.float32),
            pltpu.VMEM((_CW,), jnp.int32),
            pltpu.VMEM((_CW,), jnp.float32),
            pltpu.VMEM((16,), jnp.float32),
            pltpu.SemaphoreType.DMA,
            pltpu.SemaphoreType.DMA,
        ],
    )
    def compact(x_hbm, t_hbm, oi_hbm, ov_hbm,
                buf0, buf1, ibuf, vbuf, tvec, sem0, sem1):
        cid = lax.axis_index("c")
        sid = lax.axis_index("s")
        wid = sid * 2 + cid
        bufs = (buf0, buf1)
        sems = (sem0, sem1)
        lanes = lax.iota(jnp.int32, 16)
        for rr in range(rows_per_w):
            row = wid * rows_per_w + rr
            rbase = pl.multiple_of(row * v, 8)
            pltpu.sync_copy(t_hbm.at[pl.ds(pl.multiple_of(row * 16, 8), 16)],
                            tvec)
            t_vec = tvec[...]
            copies = [None, None]
            copies[0] = pltpu.async_copy(
                x_hbm.at[pl.ds(rbase, _CHUNK)], bufs[0], sems[0])

            def fill(j, _):
                ibuf[pl.ds(j * 16, 16)] = jnp.zeros((16,), jnp.int32)
                vbuf[pl.ds(j * 16, 16)] = jnp.full((16,), -jnp.inf,
                                                   jnp.float32)
                return 0

            lax.fori_loop(0, _CW // 16, fill, 0)

            cnts = [jnp.full((16,), u * _SEG, jnp.int32)
                    for u in range(_NSEG)]
            for ch in range(n_chunks):
                copies[ch % 2].wait()
                if ch + 1 < n_chunks:
                    copies[(ch + 1) % 2] = pltpu.async_copy(
                        x_hbm.at[pl.ds(rbase + (ch + 1) * _CHUNK, _CHUNK)],
                        bufs[(ch + 1) % 2], sems[(ch + 1) % 2])
                buf = bufs[ch % 2]
                base0 = jnp.int32(ch * _CHUNK)

                def chain_step(vj, u, cnt, iv):
                    x = buf[pl.ds(vj * 16, 16)]
                    m = x >= t_vec
                    scn = plsc.cumsum(m.astype(jnp.int32))
                    dest = jnp.minimum(cnt + scn - 1,
                                       jnp.int32(u * _SEG + _SEG - 1))
                    plsc.store_scatter(ibuf, [dest], iv, mask=m)
                    plsc.store_scatter(vbuf, [dest], x, mask=m)
                    return cnt + plsc.all_reduce_population_count(m)

                def step(j, carry):
                    cs, iv0 = carry
                    new = []
                    for u in range(_NSEG):
                        vj = j * _NSEG + u
                        new.append(chain_step(vj, u, cs[u], iv0 + u * 16))
                    return tuple(new), iv0 + 16 * _NSEG

                cs, iv0 = plsc.parallel_loop(
                    0, main, unroll=4,
                    carry=(tuple(cnts), base0 + lanes))(step)
                cnts = list(cs)
                for u in range(rem):
                    vj = main * _NSEG + u
                    cnts[u] = chain_step(vj, u, cnts[u],
                                         base0 + vj * 16 + lanes)

            obase = pl.multiple_of(row * _CW, 8)
            pltpu.sync_copy(ibuf.at[pl.ds(0, _CW)],
                            oi_hbm.at[pl.ds(obase, _CW)])
            pltpu.sync_copy(vbuf.at[pl.ds(0, _CW)],
                            ov_hbm.at[pl.ds(obase, _CW)])

    return compact


def _finalize_body(keff_ref, ci_ref, cv_ref, thr_ref, tok_ref, *, vocab):
    keff = keff_ref[0, 0]
    ci = ci_ref[...]
    cv = cv_ref[...]
    bb, w = ci.shape
    o = _ordered_int(cv)
    t_o = _kth_largest(o, keff)

    gt = o > t_o
    eq = o == t_o
    cnt_gt = jnp.sum(gt.astype(jnp.int32), axis=1, keepdims=True)
    needed = keff - cnt_gt

    def idx_step(j, pfx):
        cand = pfx | lax.shift_left(jnp.int32(1), jnp.int32(16) - j)
        cnt = jnp.sum((eq & (ci < cand)).astype(jnp.int32), axis=1,
                      keepdims=True)
        return jnp.where(cnt <= needed - 1, cand, pfx)

    cut = lax.fori_loop(0, 17, idx_step, jnp.zeros((bb, 1), jnp.int32))
    sel = gt | (eq & (ci <= cut))
    thr_ref[...] = jnp.concatenate([t_o, cut], axis=1)

    rowv = lax.broadcasted_iota(jnp.int32, (bb, w), 0)
    p = rowv * vocab + jnp.where(sel, ci, 0)
    bits = _threefry_bits(p)
    fb = lax.shift_right_logical(bits, jnp.int32(9)) | jnp.int32(0x3F800000)
    u = lax.bitcast_convert_type(fb, jnp.float32) - jnp.float32(1.0)
    tiny = jnp.float32(jnp.finfo(jnp.float32).tiny)
    u = jnp.maximum(tiny, u * (jnp.float32(1.0) - tiny) + tiny)
    g = -jnp.log(-jnp.log(u))
    score = jnp.where(sel, cv + g, -jnp.inf)
    ms = jnp.max(score, axis=1, keepdims=True)
    tok_ref[...] = jnp.min(
        jnp.where((score == ms) & sel, ci, jnp.int32(vocab)),
        axis=1, keepdims=True)


def _probs_body(x_ref, thr_ref, probs_ref):
    x = x_ref[...]
    r, v = x.shape
    t_o = thr_ref[:, 0:1]
    cut = thr_ref[:, 1:2]
    t_f = lax.bitcast_convert_type(
        jnp.where(t_o >= 0, t_o, t_o ^ jnp.int32(0x7FFFFFFF)), jnp.float32)
    idx = lax.broadcasted_iota(jnp.int32, (r, v), 1)
    sel = (x > t_f) | ((x == t_f) & (idx <= cut))
    m = jnp.max(x, axis=1, keepdims=True)
    e = jnp.where(sel, jnp.exp(x - m), jnp.float32(0))
    zinv = jnp.float32(1.0) / jnp.sum(e, axis=1, keepdims=True)
    probs_ref[...] = e * zinv


def kernel(logits, top_k):
    b, v = logits.shape
    keff = jnp.minimum(jnp.asarray(top_k, jnp.int32),
                       jnp.int32(min(50, v))).reshape(1, 1)

    t_gm = pl.pallas_call(
        _prefilter_body,
        grid=(b // _ROWS,),
        in_specs=[
            pl.BlockSpec(memory_space=pltpu.SMEM),
            pl.BlockSpec((_ROWS, v), lambda i: (i, 0)),
        ],
        out_specs=pl.BlockSpec((_ROWS, 16), lambda i: (i, 0)),
        out_shape=jax.ShapeDtypeStruct((b, 16), jnp.float32),
    )(keff, logits)

    ci_flat, cv_flat = _make_compactor(b, v, 32)(
        logits.reshape(-1), t_gm.reshape(-1))
    cand_idx = ci_flat.reshape(b, _CW)
    cand_val = cv_flat.reshape(b, _CW)

    thr, tok = pl.pallas_call(
        functools.partial(_finalize_body, vocab=v),
        in_specs=[
            pl.BlockSpec(memory_space=pltpu.SMEM),
            pl.BlockSpec((b, _CW), lambda: (0, 0)),
            pl.BlockSpec((b, _CW), lambda: (0, 0)),
        ],
        out_specs=[
            pl.BlockSpec((b, 2), lambda: (0, 0)),
            pl.BlockSpec((b, 1), lambda: (0, 0)),
        ],
        out_shape=[
            jax.ShapeDtypeStruct((b, 2), jnp.int32),
            jax.ShapeDtypeStruct((b, 1), jnp.int32),
        ],
    )(keff, cand_idx, cand_val)

    probs = pl.pallas_call(
        _probs_body,
        grid=(b // _ROWS,),
        in_specs=[
            pl.BlockSpec((_ROWS, v), lambda i: (i, 0)),
            pl.BlockSpec((_ROWS, 2), lambda i: (i, 0)),
        ],
        out_specs=pl.BlockSpec((_ROWS, v), lambda i: (i, 0)),
        out_shape=jax.ShapeDtypeStruct((b, v), jnp.float32),
    )(logits, thr)
    return tok, probs


# speedup vs baseline: 1.0013x; 1.0013x over previous
import functools

import jax
import jax.numpy as jnp
from jax import lax
from jax.experimental import pallas as pl
from jax.experimental.pallas import tpu as pltpu
from jax.experimental.pallas import tpu_sc as plsc

_ROWS = 16
_CW = 1024
_NSEG = 4
_SEG = _CW // _NSEG
_CHUNK = 20000

_KS0 = 0
_KS1 = 42
_KS2 = _KS0 ^ _KS1 ^ 0x1BD11BDA
_ROT = ((13, 15, 26, 6), (17, 29, 16, 24))


def _threefry_bits(p):
    ks = (jnp.int32(_KS0), jnp.int32(_KS1), jnp.int32(_KS2))
    x0 = jnp.zeros_like(p) + ks[0]
    x1 = p + ks[1]
    for i in range(5):
        for r in _ROT[i % 2]:
            x0 = x0 + x1
            x1 = (lax.shift_left(x1, jnp.int32(r))
                  | lax.shift_right_logical(x1, jnp.int32(32 - r)))
            x1 = x1 ^ x0
        x0 = x0 + ks[(i + 1) % 3]
        x1 = x1 + ks[(i + 2) % 3] + jnp.int32(i + 1)
    return x0 ^ x1


def _ordered_int(x):
    xi = lax.bitcast_convert_type(x, jnp.int32)
    return jnp.where(xi >= 0, xi, xi ^ jnp.int32(0x7FFFFFFF))


def _kth_largest(o, keff, nbits=32):
    r = o.shape[0]
    min32 = jnp.int32(-2147483648)

    def bit_step(j, prefix):
        cand = prefix | lax.shift_left(jnp.int32(1), jnp.int32(31) - j)
        thr = cand ^ min32
        cnt = jnp.sum((o >= thr).astype(jnp.int32), axis=1, keepdims=True)
        return jnp.where(cnt >= keff, cand, prefix)

    prefix = lax.fori_loop(0, nbits, bit_step, jnp.zeros((r, 1), jnp.int32))
    return prefix ^ min32


def _prefilter_body(keff_ref, x_ref, tgm_ref):
    keff = keff_ref[0, 0]
    x = x_ref[...]
    v = x.shape[1]
    q = v // 8
    parts = [x[:, i * q:(i + 1) * q] for i in range(8)]
    while len(parts) > 1:
        parts = [jnp.maximum(parts[i], parts[i + 1])
                 for i in range(0, len(parts), 2)]
    t_gm = _kth_largest(_ordered_int(parts[0]), keff, nbits=20)
    t_f = lax.bitcast_convert_type(
        jnp.where(t_gm >= 0, t_gm, t_gm ^ jnp.int32(0x7FFFFFFF)),
        jnp.float32)
    tgm_ref[...] = jnp.broadcast_to(t_f, tgm_ref.shape)


def _make_compactor(b, v, n_workers):
    rows_per_w = b // n_workers
    n_chunks = v // _CHUNK
    vregs = _CHUNK // 16
    main = vregs // _NSEG
    rem = vregs - main * _NSEG
    mesh = plsc.VectorSubcoreMesh(core_axis_name="c", subcore_axis_name="s")

    @functools.partial(
        pl.kernel, mesh=mesh,
        compiler_params=pltpu.CompilerParams(needs_layout_passes=False),
        out_type=[
            jax.ShapeDtypeStruct((b * _CW,), jnp.int32),
            jax.ShapeDtypeStruct((b * _CW,), jnp.float32),
        ],
        scratch_types=[
            pltpu.VMEM((_CHUNK,), jnp.float32),
            pltpu.VMEM((_CHUNK,), jnp.float32),
            pltpu.VMEM((_CW,), jnp.int32),
            pltpu.VMEM((_CW,), jnp.float32),
            pltpu.VMEM((16,), jnp.float32),
            pltpu.SemaphoreType.DMA,
            pltpu.SemaphoreType.DMA,
        ],
    )
    def compact(x_hbm, t_hbm, oi_hbm, ov_hbm,
                buf0, buf1, ibuf, vbuf, tvec, sem0, sem1):
        cid = lax.axis_index("c")
        sid = lax.axis_index("s")
        wid = sid * 2 + cid
        bufs = (buf0, buf1)
        sems = (sem0, sem1)
        lanes = lax.iota(jnp.int32, 16)
        for rr in range(rows_per_w):
            row = wid * rows_per_w + rr
            rbase = pl.multiple_of(row * v, 8)
            pltpu.sync_copy(t_hbm.at[pl.ds(pl.multiple_of(row * 16, 8), 16)],
                            tvec)
            t_vec = tvec[...]
            copies = [None, None]
            copies[0] = pltpu.async_copy(
                x_hbm.at[pl.ds(rbase, _CHUNK)], bufs[0], sems[0])

            def fill(j, _):
                ibuf[pl.ds(j * 16, 16)] = jnp.zeros((16,), jnp.int32)
                vbuf[pl.ds(j * 16, 16)] = jnp.full((16,), -jnp.inf,
                                                   jnp.float32)
                return 0

            lax.fori_loop(0, _CW // 16, fill, 0)

            cnts = [jnp.full((16,), u * _SEG, jnp.int32)
                    for u in range(_NSEG)]
            for ch in range(n_chunks):
                copies[ch % 2].wait()
                if ch + 1 < n_chunks:
                    copies[(ch + 1) % 2] = pltpu.async_copy(
                        x_hbm.at[pl.ds(rbase + (ch + 1) * _CHUNK, _CHUNK)],
                        bufs[(ch + 1) % 2], sems[(ch + 1) % 2])
                buf = bufs[ch % 2]
                base0 = jnp.int32(ch * _CHUNK)

                def chain_step(vj, u, cnt, iv):
                    x = buf[pl.ds(vj * 16, 16)]
                    m = x >= t_vec
                    scn = plsc.cumsum(m.astype(jnp.int32))
                    dest = jnp.minimum(cnt + scn - 1,
                                       jnp.int32(u * _SEG + _SEG - 1))
                    plsc.store_scatter(ibuf, [dest], iv, mask=m)
                    plsc.store_scatter(vbuf, [dest], x, mask=m)
                    return cnt + plsc.all_reduce_population_count(m)

                def step(j, carry):
                    cs, iv0 = carry
                    new = []
                    for u in range(_NSEG):
                        vj = j * _NSEG + u
                        new.append(chain_step(vj, u, cs[u], iv0 + u * 16))
                    return tuple(new), iv0 + 16 * _NSEG

                cs, iv0 = plsc.parallel_loop(
                    0, main, unroll=2,
                    carry=(tuple(cnts), base0 + lanes))(step)
                cnts = list(cs)
                for u in range(rem):
                    vj = main * _NSEG + u
                    cnts[u] = chain_step(vj, u, cnts[u],
                                         base0 + vj * 16 + lanes)

            obase = pl.multiple_of(row * _CW, 8)
            pltpu.sync_copy(ibuf.at[pl.ds(0, _CW)],
                            oi_hbm.at[pl.ds(obase, _CW)])
            pltpu.sync_copy(vbuf.at[pl.ds(0, _CW)],
                            ov_hbm.at[pl.ds(obase, _CW)])

    return compact


def _finalize_body(keff_ref, ci_ref, cv_ref, thr_ref, tok_ref, *, vocab):
    keff = keff_ref[0, 0]
    ci = ci_ref[...]
    cv = cv_ref[...]
    bb, w = ci.shape
    o = _ordered_int(cv)
    t_o = _kth_largest(o, keff)

    gt = o > t_o
    eq = o == t_o
    cnt_gt = jnp.sum(gt.astype(jnp.int32), axis=1, keepdims=True)
    needed = keff - cnt_gt

    def idx_step(j, pfx):
        cand = pfx | lax.shift_left(jnp.int32(1), jnp.int32(16) - j)
        cnt = jnp.sum((eq & (ci < cand)).astype(jnp.int32), axis=1,
                      keepdims=True)
        return jnp.where(cnt <= needed - 1, cand, pfx)

    cut = lax.fori_loop(0, 17, idx_step, jnp.zeros((bb, 1), jnp.int32))
    sel = gt | (eq & (ci <= cut))
    thr_ref[...] = jnp.concatenate([t_o, cut], axis=1)

    rowv = lax.broadcasted_iota(jnp.int32, (bb, w), 0)
    p = rowv * vocab + jnp.where(sel, ci, 0)
    bits = _threefry_bits(p)
    fb = lax.shift_right_logical(bits, jnp.int32(9)) | jnp.int32(0x3F800000)
    u = lax.bitcast_convert_type(fb, jnp.float32) - jnp.float32(1.0)
    tiny = jnp.float32(jnp.finfo(jnp.float32).tiny)
    u = jnp.maximum(tiny, u * (jnp.float32(1.0) - tiny) + tiny)
    g = -jnp.log(-jnp.log(u))
    score = jnp.where(sel, cv + g, -jnp.inf)
    ms = jnp.max(score, axis=1, keepdims=True)
    tok_ref[...] = jnp.min(
        jnp.where((score == ms) & sel, ci, jnp.int32(vocab)),
        axis=1, keepdims=True)


def _probs_body(x_ref, thr_ref, probs_ref):
    x = x_ref[...]
    r, v = x.shape
    t_o = thr_ref[:, 0:1]
    cut = thr_ref[:, 1:2]
    t_f = lax.bitcast_convert_type(
        jnp.where(t_o >= 0, t_o, t_o ^ jnp.int32(0x7FFFFFFF)), jnp.float32)
    idx = lax.broadcasted_iota(jnp.int32, (r, v), 1)
    sel = (x > t_f) | ((x == t_f) & (idx <= cut))
    m = jnp.max(x, axis=1, keepdims=True)
    e = jnp.where(sel, jnp.exp(x - m), jnp.float32(0))
    zinv = jnp.float32(1.0) / jnp.sum(e, axis=1, keepdims=True)
    probs_ref[...] = e * zinv


def kernel(logits, top_k):
    b, v = logits.shape
    keff = jnp.minimum(jnp.asarray(top_k, jnp.int32),
                       jnp.int32(min(50, v))).reshape(1, 1)

    t_gm = pl.pallas_call(
        _prefilter_body,
        grid=(b // _ROWS,),
        in_specs=[
            pl.BlockSpec(memory_space=pltpu.SMEM),
            pl.BlockSpec((_ROWS, v), lambda i: (i, 0)),
        ],
        out_specs=pl.BlockSpec((_ROWS, 16), lambda i: (i, 0)),
        out_shape=jax.ShapeDtypeStruct((b, 16), jnp.float32),
    )(keff, logits)

    ci_flat, cv_flat = _make_compactor(b, v, 32)(
        logits.reshape(-1), t_gm.reshape(-1))
    cand_idx = ci_flat.reshape(b, _CW)
    cand_val = cv_flat.reshape(b, _CW)

    thr, tok = pl.pallas_call(
        functools.partial(_finalize_body, vocab=v),
        in_specs=[
            pl.BlockSpec(memory_space=pltpu.SMEM),
            pl.BlockSpec((b, _CW), lambda: (0, 0)),
            pl.BlockSpec((b, _CW), lambda: (0, 0)),
        ],
        out_specs=[
            pl.BlockSpec((b, 2), lambda: (0, 0)),
            pl.BlockSpec((b, 1), lambda: (0, 0)),
        ],
        out_shape=[
            jax.ShapeDtypeStruct((b, 2), jnp.int32),
            jax.ShapeDtypeStruct((b, 1), jnp.int32),
        ],
    )(keff, cand_idx, cand_val)

    probs = pl.pallas_call(
        _probs_body,
        grid=(b // _ROWS,),
        in_specs=[
            pl.BlockSpec((_ROWS, v), lambda i: (i, 0)),
            pl.BlockSpec((_ROWS, 2), lambda i: (i, 0)),
        ],
        out_specs=pl.BlockSpec((_ROWS, v), lambda i: (i, 0)),
        out_shape=jax.ShapeDtypeStruct((b, v), jnp.float32),
    )(logits, thr)
    return tok, probs
